# baseline (speedup 1.0000x reference)
import jax
import jax.numpy as jnp
from jax import lax
from jax.experimental import pallas as pl
from jax.experimental.pallas import tpu as pltpu
from jax.experimental.pallas import tpu_sc as plsc

NC = 2
NS = 16
L = 16
NW = NC * NS

E = 320000
D = 128
EPW = E // NW
C = 80
NCHUNK = EPW // C


def _decoder_body(z_hbm, src_hbm, dst_hbm, out_hbm,
                  sidx, didx, srows, drows, outv, sem0, sem1):
    wid = lax.axis_index("s") * NC + lax.axis_index("c")
    lanes = lax.iota(jnp.int32, L)

    @pl.loop(0, NCHUNK)
    def _chunk(c):
        base = wid * EPW + c * C
        pltpu.sync_copy(src_hbm.at[pl.ds(base, C)], sidx)
        pltpu.sync_copy(dst_hbm.at[pl.ds(base, C)], didx)
        cp0 = pltpu.async_copy(z_hbm.at[sidx], srows, sem0)
        cp1 = pltpu.async_copy(z_hbm.at[didx], drows, sem1)
        cp0.wait()
        cp1.wait()
        nb = C // L
        zero = jnp.zeros((L,), jnp.float32)

        @pl.loop(0, D, init_carry=(zero,) * nb, unroll=4)
        def _k(k, accs):
            cols = jnp.full((L,), k, jnp.int32)
            out = []
            for b in range(nb):
                rows = lanes + b * L
                vs = plsc.load_gather(srows, [rows, cols])
                vd = plsc.load_gather(drows, [rows, cols])
                out.append(accs[b] + vs * vd)
            return tuple(out)

        for b in range(nb):
            outv[pl.ds(b * L, L)] = 1.0 / (1.0 + jnp.exp(-_k[b]))
        pltpu.sync_copy(outv, out_hbm.at[pl.ds(base, C)])


@jax.jit
def _run(z, src, dst):
    mesh = plsc.VectorSubcoreMesh(
        core_axis_name="c", subcore_axis_name="s",
        num_cores=NC, num_subcores=NS)
    f = pl.kernel(
        _decoder_body,
        out_type=jax.ShapeDtypeStruct((E,), jnp.float32),
        mesh=mesh,
        scratch_types=[
            pltpu.VMEM((C,), jnp.int32),
            pltpu.VMEM((C,), jnp.int32),
            pltpu.VMEM((C, D), jnp.float32),
            pltpu.VMEM((C, D), jnp.float32),
            pltpu.VMEM((C,), jnp.float32),
            pltpu.SemaphoreType.DMA,
            pltpu.SemaphoreType.DMA,
        ],
        compiler_params=pltpu.CompilerParams(needs_layout_passes=False),
    )
    return f(z, src, dst)


def kernel(z, edge_index):
    ei = edge_index.astype(jnp.int32)
    return _run(z, ei[0], ei[1])


# speedup vs baseline: 2.1571x; 2.1571x over previous
import jax
import jax.numpy as jnp
from jax import lax
from jax.experimental import pallas as pl
from jax.experimental.pallas import tpu as pltpu
from jax.experimental.pallas import tpu_sc as plsc

NC = 2
NS = 16
L = 16
NW = NC * NS

E = 320000
D = 128
EPW = E // NW
C = 80
NCHUNK = EPW // C


def _decoder_body(z_hbm, src_hbm, dst_hbm, out_hbm,
                  sidx, didx, srows, drows, outv, sem0, sem1):
    wid = lax.axis_index("s") * NC + lax.axis_index("c")
    lanes = lax.iota(jnp.int32, L)

    @pl.loop(0, NCHUNK)
    def _chunk(c):
        base = wid * EPW + c * C
        pltpu.sync_copy(src_hbm.at[pl.ds(base, C)], sidx)
        pltpu.sync_copy(dst_hbm.at[pl.ds(base, C)], didx)
        cp0 = pltpu.async_copy(z_hbm.at[sidx], srows, sem0)
        cp1 = pltpu.async_copy(z_hbm.at[didx], drows, sem1)
        cp0.wait()
        cp1.wait()
        nb = C // L
        zero = jnp.zeros((L,), jnp.float32)
        for b in range(nb):
            res = zero
            for e in range(L):
                row = b * L + e
                acc = srows[row, pl.ds(0, L)] * drows[row, pl.ds(0, L)]
                for j in range(1, D // L):
                    acc = acc + (srows[row, pl.ds(j * L, L)]
                                 * drows[row, pl.ds(j * L, L)])
                s = jnp.sum(acc)
                res = jnp.where(lanes == e, s, res)
            outv[pl.ds(b * L, L)] = 1.0 / (1.0 + jnp.exp(-res))
        pltpu.sync_copy(outv, out_hbm.at[pl.ds(base, C)])


@jax.jit
def _run(z, src, dst):
    mesh = plsc.VectorSubcoreMesh(
        core_axis_name="c", subcore_axis_name="s",
        num_cores=NC, num_subcores=NS)
    f = pl.kernel(
        _decoder_body,
        out_type=jax.ShapeDtypeStruct((E,), jnp.float32),
        mesh=mesh,
        scratch_types=[
            pltpu.VMEM((C,), jnp.int32),
            pltpu.VMEM((C,), jnp.int32),
            pltpu.VMEM((C, D), jnp.float32),
            pltpu.VMEM((C, D), jnp.float32),
            pltpu.VMEM((C,), jnp.float32),
            pltpu.SemaphoreType.DMA,
            pltpu.SemaphoreType.DMA,
        ],
        compiler_params=pltpu.CompilerParams(needs_layout_passes=False),
    )
    return f(z, src, dst)


def kernel(z, edge_index):
    ei = edge_index.astype(jnp.int32)
    return _run(z, ei[0], ei[1])


# speedup vs baseline: 4.0853x; 1.8939x over previous
import jax
import jax.numpy as jnp
from jax import lax
from jax.experimental import pallas as pl
from jax.experimental.pallas import tpu as pltpu
from jax.experimental.pallas import tpu_sc as plsc

NC = 2
NS = 16
L = 16
NW = NC * NS

E = 320000
D = 128
EPW = E // NW
C = 80
NCHUNK = EPW // C


def _decoder_body(z_hbm, src_hbm, dst_hbm, out_hbm,
                  sidx, didx, outv,
                  srows0, srows1, drows0, drows1,
                  sg0, sg1, sd0, sd1):
    wid = lax.axis_index("s") * NC + lax.axis_index("c")
    base0 = wid * EPW
    lanes = lax.iota(jnp.int32, L)
    srows = (srows0, srows1)
    drows = (drows0, drows1)
    sg = (sg0, sg1)
    sd = (sd0, sd1)

    pltpu.sync_copy(src_hbm.at[pl.ds(base0, EPW)], sidx)
    pltpu.sync_copy(dst_hbm.at[pl.ds(base0, EPW)], didx)

    def issue(c, p):
        pltpu.async_copy(z_hbm.at[sidx.at[pl.ds(c * C, C)]], srows[p], sg[p])
        pltpu.async_copy(z_hbm.at[didx.at[pl.ds(c * C, C)]], drows[p], sd[p])

    def wait(p):
        pltpu.make_async_copy(
            z_hbm.at[sidx.at[pl.ds(0, C)]], srows[p], sg[p]).wait()
        pltpu.make_async_copy(
            z_hbm.at[didx.at[pl.ds(0, C)]], drows[p], sd[p]).wait()

    def compute(c, p):
        sr, dr = srows[p], drows[p]

        @pl.loop(0, C // L)
        def _blk(b):
            res = jnp.zeros((L,), jnp.float32)
            for e in range(L):
                row = b * L + e
                acc = sr[row, pl.ds(0, L)] * dr[row, pl.ds(0, L)]
                for j in range(1, D // L):
                    acc = acc + (sr[row, pl.ds(j * L, L)]
                                 * dr[row, pl.ds(j * L, L)])
                s = jnp.sum(acc)
                res = jnp.where(lanes == e, s, res)
            outv[pl.ds(c * C + b * L, L)] = 1.0 / (1.0 + jnp.exp(-res))

    issue(0, 0)

    @pl.loop(0, (NCHUNK - 1) // 2)
    def _pair(t):
        c0 = 2 * t
        wait(0)
        issue(c0 + 1, 1)
        compute(c0, 0)
        wait(1)
        issue(c0 + 2, 0)
        compute(c0 + 1, 1)

    wait(0)
    compute(NCHUNK - 1, 0)
    pltpu.sync_copy(outv, out_hbm.at[pl.ds(base0, EPW)])


@jax.jit
def _run(z, src, dst):
    mesh = plsc.VectorSubcoreMesh(
        core_axis_name="c", subcore_axis_name="s",
        num_cores=NC, num_subcores=NS)
    f = pl.kernel(
        _decoder_body,
        out_type=jax.ShapeDtypeStruct((E,), jnp.float32),
        mesh=mesh,
        scratch_types=[
            pltpu.VMEM((EPW,), jnp.int32),
            pltpu.VMEM((EPW,), jnp.int32),
            pltpu.VMEM((EPW,), jnp.float32),
            pltpu.VMEM((C, D), jnp.float32),
            pltpu.VMEM((C, D), jnp.float32),
            pltpu.VMEM((C, D), jnp.float32),
            pltpu.VMEM((C, D), jnp.float32),
            pltpu.SemaphoreType.DMA,
            pltpu.SemaphoreType.DMA,
            pltpu.SemaphoreType.DMA,
            pltpu.SemaphoreType.DMA,
        ],
        compiler_params=pltpu.CompilerParams(needs_layout_passes=False),
    )
    return f(z, src, dst)


def kernel(z, edge_index):
    ei = edge_index.astype(jnp.int32)
    return _run(z, ei[0], ei[1])


# speedup vs baseline: 4.4617x; 1.0921x over previous
import jax
import jax.numpy as jnp
from jax import lax
from jax.experimental import pallas as pl
from jax.experimental.pallas import tpu as pltpu
from jax.experimental.pallas import tpu_sc as plsc

NC = 2
NS = 16
L = 16
NW = NC * NS

E = 320000
D = 128
EPW = E // NW
C = 80
NCHUNK = EPW // C


def _decoder_body(z_hbm, src_hbm, dst_hbm, out_hbm,
                  sidx, didx, outv,
                  srows0, srows1, drows0, drows1,
                  sg0, sg1, sd0, sd1):
    wid = lax.axis_index("s") * NC + lax.axis_index("c")
    base0 = wid * EPW
    lanes = lax.iota(jnp.int32, L)
    srows = (srows0, srows1)
    drows = (drows0, drows1)
    sg = (sg0, sg1)
    sd = (sd0, sd1)

    pltpu.sync_copy(src_hbm.at[pl.ds(base0, EPW)], sidx)
    pltpu.sync_copy(dst_hbm.at[pl.ds(base0, EPW)], didx)

    def issue(c, p):
        pltpu.async_copy(z_hbm.at[sidx.at[pl.ds(c * C, C)]], srows[p], sg[p])
        pltpu.async_copy(z_hbm.at[didx.at[pl.ds(c * C, C)]], drows[p], sd[p])

    def wait(p):
        pltpu.make_async_copy(
            z_hbm.at[sidx.at[pl.ds(0, C)]], srows[p], sg[p]).wait()
        pltpu.make_async_copy(
            z_hbm.at[didx.at[pl.ds(0, C)]], drows[p], sd[p]).wait()

    def compute(c, p):
        sr, dr = srows[p], drows[p]

        @pl.loop(0, C // L)
        def _blk(b):
            res = jnp.zeros((L,), jnp.float32)
            for e in range(L):
                row = b * L + e
                p = [sr[row, pl.ds(j * L, L)] * dr[row, pl.ds(j * L, L)]
                     for j in range(D // L)]
                while len(p) > 1:
                    p = [p[i] + p[i + 1] for i in range(0, len(p) - 1, 2)] \
                        + ([p[-1]] if len(p) % 2 else [])
                s = jnp.sum(p[0])
                res = jnp.where(lanes == e, s, res)
            outv[pl.ds(c * C + b * L, L)] = 1.0 / (1.0 + jnp.exp(-res))

    issue(0, 0)

    @pl.loop(0, (NCHUNK - 1) // 2)
    def _pair(t):
        c0 = 2 * t
        wait(0)
        issue(c0 + 1, 1)
        compute(c0, 0)
        wait(1)
        issue(c0 + 2, 0)
        compute(c0 + 1, 1)

    wait(0)
    compute(NCHUNK - 1, 0)
    pltpu.sync_copy(outv, out_hbm.at[pl.ds(base0, EPW)])


@jax.jit
def _run(z, src, dst):
    mesh = plsc.VectorSubcoreMesh(
        core_axis_name="c", subcore_axis_name="s",
        num_cores=NC, num_subcores=NS)
    f = pl.kernel(
        _decoder_body,
        out_type=jax.ShapeDtypeStruct((E,), jnp.float32),
        mesh=mesh,
        scratch_types=[
            pltpu.VMEM((EPW,), jnp.int32),
            pltpu.VMEM((EPW,), jnp.int32),
            pltpu.VMEM((EPW,), jnp.float32),
            pltpu.VMEM((C, D), jnp.float32),
            pltpu.VMEM((C, D), jnp.float32),
            pltpu.VMEM((C, D), jnp.float32),
            pltpu.VMEM((C, D), jnp.float32),
            pltpu.SemaphoreType.DMA,
            pltpu.SemaphoreType.DMA,
            pltpu.SemaphoreType.DMA,
            pltpu.SemaphoreType.DMA,
        ],
        compiler_params=pltpu.CompilerParams(needs_layout_passes=False),
    )
    return f(z, src, dst)


def kernel(z, edge_index):
    ei = edge_index.astype(jnp.int32)
    return _run(z, ei[0], ei[1])


# speedup vs baseline: 4.5411x; 1.0178x over previous
import jax
import jax.numpy as jnp
import numpy as np
from jax import lax
from jax.experimental import pallas as pl
from jax.experimental.pallas import tpu as pltpu
from jax.experimental.pallas import tpu_sc as plsc

NC = 2
NS = 16
L = 16
NW = NC * NS

E = 320000
D = 128
EPW = E // NW
C = 80
NCHUNK = EPW // C


def _decoder_body(z_hbm, src_hbm, dst_hbm, out_hbm,
                  sidx, didx, outv,
                  srows0, srows1, drows0, drows1,
                  sg0, sg1, sd0, sd1):
    wid = lax.axis_index("s") * NC + lax.axis_index("c")
    base0 = wid * EPW
    lanes = lax.iota(jnp.int32, L)
    srows = (srows0, srows1)
    drows = (drows0, drows1)
    sg = (sg0, sg1)
    sd = (sd0, sd1)

    pltpu.sync_copy(src_hbm.at[pl.ds(base0, EPW)], sidx)
    pltpu.sync_copy(dst_hbm.at[pl.ds(base0, EPW)], didx)

    @pl.loop(0, EPW // L)
    def _zero(i):
        outv[pl.ds(i * L, L)] = jnp.zeros((L,), jnp.float32)

    def issue(c, p):
        pltpu.async_copy(z_hbm.at[sidx.at[pl.ds(c * C, C)]], srows[p], sg[p])
        pltpu.async_copy(z_hbm.at[didx.at[pl.ds(c * C, C)]], drows[p], sd[p])

    def wait(p):
        pltpu.make_async_copy(
            z_hbm.at[sidx.at[pl.ds(0, C)]], srows[p], sg[p]).wait()
        pltpu.make_async_copy(
            z_hbm.at[didx.at[pl.ds(0, C)]], drows[p], sd[p]).wait()

    def compute(c, p):
        sr, dr = srows[p], drows[p]

        @pl.loop(0, C // L)
        def _blk(b):
            out_base = c * C + b * L
            for e in range(L):
                row = b * L + e
                pr = [sr[row, pl.ds(j * L, L)] * dr[row, pl.ds(j * L, L)]
                      for j in range(D // L)]
                while len(pr) > 1:
                    pr = [pr[k] + pr[k + 1] for k in range(0, len(pr) - 1, 2)] \
                        + ([pr[-1]] if len(pr) % 2 else [])
                plsc.addupdate_scatter(
                    outv, [jnp.full((L,), out_base + e, jnp.int32)], pr[0])
            acc = outv[pl.ds(out_base, L)]
            outv[pl.ds(out_base, L)] = 1.0 / (1.0 + jnp.exp(-acc))

    issue(0, 0)

    @pl.loop(0, (NCHUNK - 1) // 2)
    def _pair(t):
        c0 = 2 * t
        wait(0)
        issue(c0 + 1, 1)
        compute(c0, 0)
        wait(1)
        issue(c0 + 2, 0)
        compute(c0 + 1, 1)

    wait(0)
    compute(NCHUNK - 1, 0)
    pltpu.sync_copy(outv, out_hbm.at[pl.ds(base0, EPW)])


@jax.jit
def _run(z, src, dst):
    mesh = plsc.VectorSubcoreMesh(
        core_axis_name="c", subcore_axis_name="s",
        num_cores=NC, num_subcores=NS)
    f = pl.kernel(
        _decoder_body,
        out_type=jax.ShapeDtypeStruct((E,), jnp.float32),
        mesh=mesh,
        scratch_types=[
            pltpu.VMEM((EPW,), jnp.int32),
            pltpu.VMEM((EPW,), jnp.int32),
            pltpu.VMEM((EPW,), jnp.float32),
            pltpu.VMEM((C, D), jnp.float32),
            pltpu.VMEM((C, D), jnp.float32),
            pltpu.VMEM((C, D), jnp.float32),
            pltpu.VMEM((C, D), jnp.float32),
            pltpu.SemaphoreType.DMA,
            pltpu.SemaphoreType.DMA,
            pltpu.SemaphoreType.DMA,
            pltpu.SemaphoreType.DMA,
        ],
        compiler_params=pltpu.CompilerParams(needs_layout_passes=False),
    )
    return f(z, src, dst)


def kernel(z, edge_index):
    ei = edge_index.astype(jnp.int32)
    return _run(z, ei[0], ei[1])
